# baseline (speedup 1.0000x reference)
from itertools import product

import jax
import jax.numpy as jnp
from jax import lax
from jax.experimental import pallas as pl
from jax.experimental.pallas import tpu as pltpu
from jax.experimental.pallas import tpu_sc as plsc

NC, NS, L = 2, 16, 16
NW = NC * NS
NPTS = 16384
PPT = NPTS // NW
CHUNK = 16
NCHUNK = PPT // CHUNK

SCALES = ((32, 64), (16, 128), (8, 256))
CTOT = sum(c for _, c in SCALES)
COFF = (0, 64, 192)
CORNERS = tuple(product((0, 1), repeat=3))


def _idx_weights(v_ref, off, G):
  v = v_ref[pl.ds(off, CHUNK)] * float(G)
  v = jnp.minimum(jnp.maximum(v, 0.01), float(G) - 1.01)
  i1 = v.astype(jnp.int32)
  w = v - i1.astype(jnp.float32)
  w2 = jnp.where(w == 0.0, 0.0, 1.0 - w)
  return i1, (w2, w)


def _sc_body(xs_hbm, ys_hbm, zs_hbm, t1, t2, t3, out_hbm,
             xs_v, ys_v, zs_v, idx_b, w_b, rows_b, out_acc, gsems, osem):
  wid = lax.axis_index("s") * NC + lax.axis_index("c")
  base = wid * PPT
  pltpu.sync_copy(xs_hbm.at[pl.ds(base, PPT)], xs_v)
  pltpu.sync_copy(ys_hbm.at[pl.ds(base, PPT)], ys_v)
  pltpu.sync_copy(zs_hbm.at[pl.ds(base, PPT)], zs_v)
  tables = (t1, t2, t3)

  def stage(c, buf):
    off = c * CHUNK
    for s, (G, C) in enumerate(SCALES):
      x1, ux = _idx_weights(xs_v, off, G)
      y1, vy = _idx_weights(ys_v, off, G)
      z1, tz = _idx_weights(zs_v, off, G)
      ibase = x1 * (G * G) + y1 * G + z1
      for k, (a, b, cc) in enumerate(CORNERS):
        idx_b[buf][s][pl.ds(k * L, L)] = ibase + (a * G * G + b * G + cc)
        w_b[buf][s][pl.ds(k * L, L)] = (ux[a] * vy[b]) * tz[cc]
      pltpu.async_copy(tables[s].at[idx_b[buf][s]], rows_b[buf][s],
                       gsems[buf][s])

  def blend(c, buf):
    for s in range(len(SCALES)):
      pltpu.make_async_copy(tables[s].at[idx_b[buf][s]], rows_b[buf][s],
                            gsems[buf][s]).wait()

    @plsc.parallel_loop(0, CHUNK, unroll=4)
    def pt_body(p):
      for s, (G, C) in enumerate(SCALES):
        ws = [w_b[buf][s][pl.ds(k * L + p, L)][0] for k in range(8)]
        rows = rows_b[buf][s]
        for j in range(SCALES[s][1] // L):
          r = [rows[k * CHUNK + p, pl.ds(j * L, L)] for k in range(8)]
          m = [ws[k] * r[k] for k in range(8)]
          acc = ((m[0] + m[1]) + (m[2] + m[3])) + \
                ((m[4] + m[5]) + (m[6] + m[7]))
          out_acc[p, pl.ds(COFF[s] + j * L, L)] = acc

  stage(0, 0)

  def body2(c2, _):
    for par in (0, 1):
      c = 2 * c2 + par
      @pl.when(c + 1 < NCHUNK)
      def _():
        stage(c + 1, 1 - par)
      @pl.when(c > 0)
      def _():
        pltpu.make_async_copy(
            out_acc, out_hbm.at[pl.ds(base + (c - 1) * CHUNK, CHUNK)],
            osem).wait()
      blend(c, par)
      pltpu.async_copy(out_acc, out_hbm.at[pl.ds(base + c * CHUNK, CHUNK)],
                       osem)
    return 0

  lax.fori_loop(0, NCHUNK // 2, body2, 0)
  pltpu.make_async_copy(
      out_acc, out_hbm.at[pl.ds(base + (NCHUNK - 1) * CHUNK, CHUNK)],
      osem).wait()


@jax.jit
def kernel(features0, features1, features2, features3, features4,
           mesh_coords, mesh_features):
  del features0, features4
  f1 = features1.reshape(32 * 32 * 32, 64)
  f2 = features2.reshape(16 * 16 * 16, 128)
  f3 = features3.reshape(8 * 8 * 8, 256)
  xs = mesh_coords[0, :, 0]
  ys = mesh_coords[0, :, 1]
  zs = mesh_coords[0, :, 2]

  mesh = plsc.VectorSubcoreMesh(core_axis_name="c", subcore_axis_name="s",
                                num_cores=NC, num_subcores=NS)
  out = pl.kernel(
      _sc_body,
      out_type=jax.ShapeDtypeStruct((NPTS, CTOT), jnp.float32),
      mesh=mesh,
      compiler_params=pltpu.CompilerParams(use_tc_tiling_on_sc=False),
      scratch_types=[
          pltpu.VMEM((PPT,), jnp.float32),
          pltpu.VMEM((PPT,), jnp.float32),
          pltpu.VMEM((PPT,), jnp.float32),
          [[pltpu.VMEM((8 * L,), jnp.int32) for _ in SCALES]
           for _ in range(2)],
          [[pltpu.VMEM((9 * L,), jnp.float32) for _ in SCALES]
           for _ in range(2)],
          [[pltpu.VMEM((8 * CHUNK, C), jnp.float32) for _, C in SCALES]
           for _ in range(2)],
          pltpu.VMEM((CHUNK, CTOT), jnp.float32),
          [[pltpu.SemaphoreType.DMA for _ in SCALES] for _ in range(2)],
          pltpu.SemaphoreType.DMA,
      ],
  )(xs, ys, zs, f1, f2, f3)
  return jnp.concatenate([out[None], mesh_features], axis=-1)


# speedup vs baseline: 1.3484x; 1.3484x over previous
from itertools import product

import jax
import jax.numpy as jnp
from jax import lax
from jax.experimental import pallas as pl
from jax.experimental.pallas import tpu as pltpu
from jax.experimental.pallas import tpu_sc as plsc

NC, NS, L = 2, 16, 16
NW = NC * NS
NPTS = 16384
PPT = NPTS // NW
CHUNK = 16
NCHUNK = PPT // CHUNK

SCALES = ((32, 64), (16, 128), (8, 256))
CTOT = sum(c for _, c in SCALES)
COFF = (0, 64, 192)
CORNERS = tuple(product((0, 1), repeat=3))


def _idx_weights(v_ref, off, G):
  v = v_ref[pl.ds(off, CHUNK)] * float(G)
  v = jnp.minimum(jnp.maximum(v, 0.01), float(G) - 1.01)
  i1 = v.astype(jnp.int32)
  w = v - i1.astype(jnp.float32)
  w2 = jnp.where(w == 0.0, 0.0, 1.0 - w)
  return i1, (w2, w)


def _sc_body(xs_hbm, ys_hbm, zs_hbm, t1, t2, t3, out_hbm,
             xs_v, ys_v, zs_v, idx_b, w_b, rows_b, out_acc, gsems, osem):
  wid = lax.axis_index("s") * NC + lax.axis_index("c")
  base = wid * PPT
  pltpu.sync_copy(xs_hbm.at[pl.ds(base, PPT)], xs_v)
  pltpu.sync_copy(ys_hbm.at[pl.ds(base, PPT)], ys_v)
  pltpu.sync_copy(zs_hbm.at[pl.ds(base, PPT)], zs_v)
  tables = (t1, t2, t3)

  def stage(c, buf):
    off = c * CHUNK
    for s, (G, C) in enumerate(SCALES):
      x1, ux = _idx_weights(xs_v, off, G)
      y1, vy = _idx_weights(ys_v, off, G)
      z1, tz = _idx_weights(zs_v, off, G)
      ibase = x1 * (G * G) + y1 * G + z1
      for k, (a, b, cc) in enumerate(CORNERS):
        idx_b[buf][s][pl.ds(k * L, L)] = ibase + (a * G * G + b * G + cc)
        w_b[buf][s][pl.ds(k * L, L)] = (ux[a] * vy[b]) * tz[cc]
      pltpu.async_copy(tables[s].at[idx_b[buf][s]], rows_b[buf][s],
                       gsems[buf][s])

  def blend(c, buf):
    for s in range(len(SCALES)):
      pltpu.make_async_copy(tables[s].at[idx_b[buf][s]], rows_b[buf][s],
                            gsems[buf][s]).wait()

    @plsc.parallel_loop(0, CHUNK, unroll=2)
    def pt_body(p):
      for s, (G, C) in enumerate(SCALES):
        ws = [w_b[buf][s][pl.ds(k * L + p, L)][0] for k in range(8)]
        rows = rows_b[buf][s]
        for j in range(SCALES[s][1] // L):
          r = [rows[k * CHUNK + p, pl.ds(j * L, L)] for k in range(8)]
          m = [ws[k] * r[k] for k in range(8)]
          acc = ((m[0] + m[1]) + (m[2] + m[3])) + \
                ((m[4] + m[5]) + (m[6] + m[7]))
          out_acc[p, pl.ds(COFF[s] + j * L, L)] = acc

  stage(0, 0)

  def body2(c2, _):
    for par in (0, 1):
      c = 2 * c2 + par
      @pl.when(c + 1 < NCHUNK)
      def _():
        stage(c + 1, 1 - par)
      @pl.when(c > 0)
      def _():
        pltpu.make_async_copy(
            out_acc, out_hbm.at[pl.ds(base + (c - 1) * CHUNK, CHUNK)],
            osem).wait()
      blend(c, par)
      pltpu.async_copy(out_acc, out_hbm.at[pl.ds(base + c * CHUNK, CHUNK)],
                       osem)
    return 0

  lax.fori_loop(0, NCHUNK // 2, body2, 0)
  pltpu.make_async_copy(
      out_acc, out_hbm.at[pl.ds(base + (NCHUNK - 1) * CHUNK, CHUNK)],
      osem).wait()


@jax.jit
def kernel(features0, features1, features2, features3, features4,
           mesh_coords, mesh_features):
  del features0, features4
  f1 = features1.reshape(32 * 32 * 32, 64)
  f2 = features2.reshape(16 * 16 * 16, 128)
  f3 = features3.reshape(8 * 8 * 8, 256)
  xs = mesh_coords[0, :, 0]
  ys = mesh_coords[0, :, 1]
  zs = mesh_coords[0, :, 2]

  mesh = plsc.VectorSubcoreMesh(core_axis_name="c", subcore_axis_name="s",
                                num_cores=NC, num_subcores=NS)
  out = pl.kernel(
      _sc_body,
      out_type=jax.ShapeDtypeStruct((NPTS, CTOT), jnp.float32),
      mesh=mesh,
      compiler_params=pltpu.CompilerParams(use_tc_tiling_on_sc=False),
      scratch_types=[
          pltpu.VMEM((PPT,), jnp.float32),
          pltpu.VMEM((PPT,), jnp.float32),
          pltpu.VMEM((PPT,), jnp.float32),
          [[pltpu.VMEM((8 * L,), jnp.int32) for _ in SCALES]
           for _ in range(2)],
          [[pltpu.VMEM((9 * L,), jnp.float32) for _ in SCALES]
           for _ in range(2)],
          [[pltpu.VMEM((8 * CHUNK, C), jnp.float32) for _, C in SCALES]
           for _ in range(2)],
          pltpu.VMEM((CHUNK, CTOT), jnp.float32),
          [[pltpu.SemaphoreType.DMA for _ in SCALES] for _ in range(2)],
          pltpu.SemaphoreType.DMA,
      ],
  )(xs, ys, zs, f1, f2, f3)
  return jnp.concatenate([out[None], mesh_features], axis=-1)
